# baseline (speedup 1.0000x reference)
import functools

import jax
import jax.numpy as jnp
from jax import lax
from jax.experimental import pallas as pl
from jax.experimental.pallas import tpu as pltpu
from jax.experimental.pallas import tpu_sc as plsc

_BN_EPS = 1e-5

_NC = 2
_NS = 16
_L = 16
_EB = 128


def _segment_sum(tbl2, src_p, dst_p, n_nodes, d=128):
  n_edges = src_p.shape[0]
  nb = n_edges // _EB
  nit = nb // _NS
  zrows = 80
  nblk = n_nodes // zrows

  mesh = plsc.VectorSubcoreMesh(core_axis_name="c", subcore_axis_name="s")

  @functools.partial(
      pl.kernel,
      out_type=jax.ShapeDtypeStruct((2 * n_nodes, d), jnp.float32),
      name="segsum_fsplit",
      mesh=mesh,
      scratch_types=[
          pltpu.VMEM((_EB,), jnp.int32),
          pltpu.VMEM((_EB,), jnp.int32),
          pltpu.VMEM((_EB,), jnp.int32),
          pltpu.VMEM((_EB, d), jnp.float32),
          pltpu.VMEM((_EB,), jnp.int32),
          pltpu.VMEM((_EB,), jnp.int32),
          pltpu.VMEM((_EB,), jnp.int32),
          pltpu.VMEM((_EB, d), jnp.float32),
          pltpu.VMEM((zrows, d), jnp.float32),
          pltpu.VMEM_SHARED((n_nodes + 8, d), jnp.float32),
          pltpu.SemaphoreType.DMA,
          pltpu.SemaphoreType.DMA,
          pltpu.SemaphoreType.DMA,
          pltpu.SemaphoreType.DMA,
      ],
  )
  def seg_kernel(tbl_ref, src_ref, dst_ref, out_ref,
                 src_v0, dst_v0, adj_v0, rows_v0,
                 src_v1, dst_v1, adj_v1, rows_v1,
                 zbuf, acc, sg0, sg1, ss0, ss1):
    c = lax.axis_index("c")
    s = lax.axis_index("s")
    zero16 = jnp.zeros((_L,), jnp.float32)
    src_vs = (src_v0, src_v1)
    dst_vs = (dst_v0, dst_v1)
    adj_vs = (adj_v0, adj_v1)
    rows_vs = (rows_v0, rows_v1)
    sgs = (sg0, sg1)
    sss = (ss0, ss1)

    @pl.loop(0, zrows)
    def _zero_zbuf(i):
      for j in range(d // _L):
        zbuf[i, pl.ds(j * _L, _L)] = zero16

    @pl.loop(s, nblk, step=_NS)
    def _zero_acc(k):
      pltpu.sync_copy(zbuf, acc.at[pl.ds(k * zrows, zrows)])

    plsc.subcore_barrier()

    off16 = jnp.full((_L,), c * n_nodes, jnp.int32)

    def fire_gather(slot, b):
      base = b * _EB
      pltpu.sync_copy(src_ref.at[pl.ds(base, _EB)], src_vs[slot])
      pltpu.sync_copy(dst_ref.at[pl.ds(base, _EB)], dst_vs[slot])
      for j in range(_EB // _L):
        sl = pl.ds(j * _L, _L)
        adj_vs[slot][sl] = src_vs[slot][sl] + off16
      pltpu.async_copy(tbl_ref.at[adj_vs[slot]], rows_vs[slot], sgs[slot])

    def wait_gather(slot):
      pltpu.make_async_copy(tbl_ref.at[pl.ds(0, _EB)],
                            rows_vs[slot], sgs[slot]).wait()

    def fire_scatter(slot):
      pltpu.async_copy(rows_vs[slot], acc.at[dst_vs[slot]], sss[slot],
                       add=True)

    def wait_scatter(slot):
      pltpu.make_async_copy(tbl_ref.at[pl.ds(0, _EB)],
                            rows_vs[slot], sss[slot]).wait()

    fire_gather(0, s)
    fire_gather(1, s + _NS)

    @pl.loop(0, (nit - 2) // 2)
    def _pairs(p):
      b0 = s + (2 * p) * _NS
      wait_gather(0)
      fire_scatter(0)
      wait_gather(1)
      fire_scatter(1)
      wait_scatter(0)
      fire_gather(0, b0 + 2 * _NS)
      wait_scatter(1)
      fire_gather(1, b0 + 3 * _NS)

    wait_gather(0)
    fire_scatter(0)
    wait_gather(1)
    fire_scatter(1)
    wait_scatter(0)
    wait_scatter(1)

    plsc.subcore_barrier()

    @pl.loop(s, nblk, step=_NS)
    def _writeback(k):
      r0 = k * zrows
      pltpu.sync_copy(acc.at[pl.ds(r0, zrows)],
                      out_ref.at[pl.ds(c * n_nodes + r0, zrows)])

  return seg_kernel(tbl2, src_p, dst_p)


def _gather_pair(a_tbl, b_tbl, src, dst):
  n_edges = src.shape[0]
  nb = n_edges // _EB
  nw = _NC * _NS
  nit = nb // nw
  d = a_tbl.shape[1]

  mesh = plsc.VectorSubcoreMesh(core_axis_name="c", subcore_axis_name="s")

  @functools.partial(
      pl.kernel,
      out_type=[
          jax.ShapeDtypeStruct((n_edges, d), jnp.uint32),
          jax.ShapeDtypeStruct((n_edges, d), jnp.uint32),
      ],
      name="gather_pair",
      mesh=mesh,
      scratch_types=[
          pltpu.VMEM((_EB,), jnp.int32),
          pltpu.VMEM((_EB,), jnp.int32),
          pltpu.VMEM((_EB, 128), jnp.uint32),
          pltpu.VMEM((_EB, 128), jnp.uint32),
          pltpu.VMEM((_EB,), jnp.int32),
          pltpu.VMEM((_EB,), jnp.int32),
          pltpu.VMEM((_EB, 128), jnp.uint32),
          pltpu.VMEM((_EB, 128), jnp.uint32),
          pltpu.SemaphoreType.DMA,
          pltpu.SemaphoreType.DMA,
          pltpu.SemaphoreType.DMA,
          pltpu.SemaphoreType.DMA,
      ],
  )
  def ga_kernel(a_ref, b_ref, src_ref, dst_ref, outa_ref, outb_ref,
                src_v0, dst_v0, ar0, br0, src_v1, dst_v1, ar1, br1,
                sg0, sg1, ss0, ss1):
    c = lax.axis_index("c")
    s = lax.axis_index("s")
    wid = s * _NC + c
    src_vs = (src_v0, src_v1)
    dst_vs = (dst_v0, dst_v1)
    ars = (ar0, ar1)
    brs = (br0, br1)
    sgs = (sg0, sg1)
    sss = (ss0, ss1)

    def fire(slot, b):
      base = b * _EB
      pltpu.sync_copy(src_ref.at[pl.ds(base, _EB)], src_vs[slot])
      pltpu.sync_copy(dst_ref.at[pl.ds(base, _EB)], dst_vs[slot])
      pltpu.async_copy(a_ref.at[src_vs[slot]], ars[slot], sgs[slot])
      pltpu.async_copy(b_ref.at[dst_vs[slot]], brs[slot], sgs[slot])

    def wait_gathers(slot):
      pltpu.make_async_copy(a_ref.at[pl.ds(0, _EB)],
                            ars[slot], sgs[slot]).wait()
      pltpu.make_async_copy(a_ref.at[pl.ds(0, _EB)],
                            brs[slot], sgs[slot]).wait()

    def store(slot, b):
      base = b * _EB
      pltpu.async_copy(ars[slot], outa_ref.at[pl.ds(base, _EB)], sss[slot])
      pltpu.async_copy(brs[slot], outb_ref.at[pl.ds(base, _EB)], sss[slot])

    def wait_stores(slot):
      pltpu.make_async_copy(a_ref.at[pl.ds(0, _EB)],
                            ars[slot], sss[slot]).wait()
      pltpu.make_async_copy(a_ref.at[pl.ds(0, _EB)],
                            brs[slot], sss[slot]).wait()

    fire(0, wid)
    fire(1, wid + nw)

    @pl.loop(0, (nit - 2) // 2)
    def _pairs(p):
      b0 = wid + (2 * p) * nw
      wait_gathers(0)
      store(0, b0)
      wait_gathers(1)
      store(1, b0 + nw)
      wait_stores(0)
      fire(0, b0 + 2 * nw)
      wait_stores(1)
      fire(1, b0 + 3 * nw)

    b_last = wid + (nit - 2) * nw
    wait_gathers(0)
    store(0, b_last)
    wait_gathers(1)
    store(1, b_last + nw)
    wait_stores(0)
    wait_stores(1)

  return ga_kernel(a_tbl, b_tbl, src, dst)


def _full2d(r, c):
  return pl.BlockSpec((r, c), lambda i: (0, 0))


def _node_proj_call(x, wnT, bn):
  n, d_in = x.shape
  h = wnT.shape[1]
  bn_rows = 1000
  grid = n // bn_rows

  def body(x_ref, w_ref, b_ref, o_ref):
    h0 = jnp.dot(x_ref[...], w_ref[...],
                 preferred_element_type=jnp.float32) + b_ref[...]
    o_ref[0] = h0[:, :128]
    o_ref[1] = h0[:, 128:]

  return pl.pallas_call(
      body,
      grid=(grid,),
      in_specs=[
          pl.BlockSpec((bn_rows, d_in), lambda i: (i, 0)),
          _full2d(d_in, h),
          _full2d(1, h),
      ],
      out_specs=pl.BlockSpec((2, bn_rows, 128), lambda i: (0, i, 0)),
      out_shape=jax.ShapeDtypeStruct((2, n, 128), jnp.float32),
  )(x, wnT, bn)


def _gin_mlp_call(h2s, agg2s, ce, waT, ba, wbT, bb, scale, shift,
                  split_out):
  n = h2s.shape[1]
  h = 256
  bn_rows = 1000
  grid = n // bn_rows

  def body(h_ref, a_ref, ce_ref, wa_ref, ba_ref, wb_ref, bb_ref,
           sc_ref, sh_ref, o_ref):
    hcat = jnp.concatenate([h_ref[0], h_ref[1]], axis=1)
    acat = jnp.concatenate([a_ref[0], a_ref[1]], axis=1)
    t = ce_ref[0, 0] * hcat + acat
    t = jnp.maximum(jnp.dot(t, wa_ref[...],
                            preferred_element_type=jnp.float32) + ba_ref[...],
                    0.0)
    t = jnp.maximum(jnp.dot(t, wb_ref[...],
                            preferred_element_type=jnp.float32) + bb_ref[...],
                    0.0)
    t = jnp.maximum(t * sc_ref[...] + sh_ref[...], 0.0)
    if split_out:
      o_ref[0] = t[:, :128]
      o_ref[1] = t[:, 128:]
    else:
      o_ref[...] = t

  if split_out:
    out_spec = pl.BlockSpec((2, bn_rows, 128), lambda i: (0, i, 0))
    out_shape = jax.ShapeDtypeStruct((2, n, 128), jnp.float32)
  else:
    out_spec = pl.BlockSpec((bn_rows, h), lambda i: (i, 0))
    out_shape = jax.ShapeDtypeStruct((n, h), jnp.float32)

  return pl.pallas_call(
      body,
      grid=(grid,),
      in_specs=[
          pl.BlockSpec((2, bn_rows, 128), lambda i: (0, i, 0)),
          pl.BlockSpec((2, bn_rows, 128), lambda i: (0, i, 0)),
          _full2d(1, 1),
          _full2d(h, h),
          _full2d(1, h),
          _full2d(h, h),
          _full2d(1, h),
          _full2d(1, h),
          _full2d(1, h),
      ],
      out_specs=out_spec,
      out_shape=out_shape,
  )(h2s, agg2s, ce, waT, ba, wbT, bb, scale, shift)


def _round_pack_bf16(even, odd):
  ue = lax.bitcast_convert_type(even, jnp.uint32)
  uo = lax.bitcast_convert_type(odd, jnp.uint32)
  ue = ue + jnp.uint32(0x7FFF) + ((ue >> 16) & jnp.uint32(1))
  uo = uo + jnp.uint32(0x7FFF) + ((uo >> 16) & jnp.uint32(1))
  return (ue >> 16) | (uo & jnp.uint32(0xFFFF0000))


def _unpack_bf16(word):
  f_even = lax.bitcast_convert_type(word << 16, jnp.float32)
  f_odd = lax.bitcast_convert_type(word & jnp.uint32(0xFFFF0000), jnp.float32)
  return f_even, f_odd


def _gin2_ab_call(h1s, agg2s, ce, waT, ba, wbT, bb, scale, shift,
                  wAT_e, wAT_o, ca_e, ca_o, wBT_e, wBT_o):
  n = h1s.shape[1]
  h = 256
  hw = h // 2
  bn_rows = 1000
  grid = n // bn_rows

  def body(h_ref, a_ref, ce_ref, wa_ref, ba_ref, wb_ref, bb_ref,
           sc_ref, sh_ref, wae_ref, wao_ref, cae_ref, cao_ref,
           wbe_ref, wbo_ref, oa_ref, ob_ref):
    hcat = jnp.concatenate([h_ref[0], h_ref[1]], axis=1)
    acat = jnp.concatenate([a_ref[0], a_ref[1]], axis=1)
    t = ce_ref[0, 0] * hcat + acat
    t = jnp.maximum(jnp.dot(t, wa_ref[...],
                            preferred_element_type=jnp.float32) + ba_ref[...],
                    0.0)
    t = jnp.maximum(jnp.dot(t, wb_ref[...],
                            preferred_element_type=jnp.float32) + bb_ref[...],
                    0.0)
    h2 = jnp.maximum(t * sc_ref[...] + sh_ref[...], 0.0)
    ae = jnp.dot(h2, wae_ref[...], preferred_element_type=jnp.float32) \
        + cae_ref[...]
    ao = jnp.dot(h2, wao_ref[...], preferred_element_type=jnp.float32) \
        + cao_ref[...]
    be = jnp.dot(h2, wbe_ref[...], preferred_element_type=jnp.float32)
    bo = jnp.dot(h2, wbo_ref[...], preferred_element_type=jnp.float32)
    oa_ref[...] = _round_pack_bf16(ae, ao)
    ob_ref[...] = _round_pack_bf16(be, bo)

  return pl.pallas_call(
      body,
      grid=(grid,),
      in_specs=[
          pl.BlockSpec((2, bn_rows, 128), lambda i: (0, i, 0)),
          pl.BlockSpec((2, bn_rows, 128), lambda i: (0, i, 0)),
          _full2d(1, 1),
          _full2d(h, h),
          _full2d(1, h),
          _full2d(h, h),
          _full2d(1, h),
          _full2d(1, h),
          _full2d(1, h),
          _full2d(h, hw),
          _full2d(h, hw),
          _full2d(1, hw),
          _full2d(1, hw),
          _full2d(h, hw),
          _full2d(h, hw),
      ],
      out_specs=[
          pl.BlockSpec((bn_rows, hw), lambda i: (i, 0)),
          pl.BlockSpec((bn_rows, hw), lambda i: (i, 0)),
      ],
      out_shape=[
          jax.ShapeDtypeStruct((n, hw), jnp.uint32),
          jax.ShapeDtypeStruct((n, hw), jnp.uint32),
      ],
  )(h1s, agg2s, ce, waT, ba, wbT, bb, scale, shift,
    wAT_e, wAT_o, ca_e, ca_o, wBT_e, wBT_o)


def _edge_head_call(ga, gb, edge_attr, wcT_p, w2T_p, b2, w3T, b3):
  e = ga.shape[0]
  be_rows = 2048
  grid = e // be_rows
  d_edge = edge_attr.shape[1]
  h = 2 * ga.shape[1]
  h2 = w2T_p.shape[1]
  c_out = w3T.shape[1]

  def body(ga_ref, gb_ref, ea_ref, wc_ref, w2_ref, b2_ref, w3_ref, b3_ref,
           o_ref):
    ec = jnp.dot(ea_ref[...], wc_ref[...], preferred_element_type=jnp.float32)
    a_even, a_odd = _unpack_bf16(ga_ref[...])
    b_even, b_odd = _unpack_bf16(gb_ref[...])
    gp = jnp.concatenate([a_even + b_even, a_odd + b_odd], axis=1)
    z1 = jnp.maximum(gp + ec, 0.0)
    z2 = jnp.maximum(jnp.dot(z1, w2_ref[...],
                             preferred_element_type=jnp.float32) + b2_ref[...],
                     0.0)
    o_ref[...] = jnp.dot(z2, w3_ref[...],
                         preferred_element_type=jnp.float32) + b3_ref[...]

  return pl.pallas_call(
      body,
      grid=(grid,),
      in_specs=[
          pl.BlockSpec((be_rows, h // 2), lambda i: (i, 0)),
          pl.BlockSpec((be_rows, h // 2), lambda i: (i, 0)),
          pl.BlockSpec((be_rows, d_edge), lambda i: (i, 0)),
          _full2d(d_edge, h),
          _full2d(h, h2),
          _full2d(1, h2),
          _full2d(h2, c_out),
          _full2d(1, c_out),
      ],
      out_specs=pl.BlockSpec((be_rows, c_out), lambda i: (i, 0)),
      out_shape=jax.ShapeDtypeStruct((e, c_out), jnp.float32),
  )(ga, gb, edge_attr, wcT_p, w2T_p, b2, w3T, b3)


def kernel(x, edge_index, edge_attr, W_node, b_node, W_edge, b_edge,
           eps1, W1a, b1a, W1b, b1b, g1, bt1,
           eps2, W2a, b2a, W2b, b2b, g2, bt2,
           Wm1, bm1, gm1, btm1, Wm2, bm2, gm2, btm2, Wm3, bm3):
  n = x.shape[0]
  h = W_node.shape[0]

  src = edge_index[0]
  dst = edge_index[1]

  n_edges = src.shape[0]
  quant = _EB * 2 * _NS
  n_pad = (-n_edges) % quant
  if (n_edges + n_pad) // _EB // _NS < 4:
    n_pad += 2 * quant
  src_p = jnp.concatenate([src, jnp.zeros((n_pad,), jnp.int32)])
  dst_p = jnp.concatenate([dst, jnp.full((n_pad,), n, jnp.int32)])

  inv = 1.0 / jnp.sqrt(jnp.float32(1.0 + _BN_EPS))
  s1 = (g1 * inv).reshape(1, h)
  s2 = (g2 * inv).reshape(1, h)
  sm1 = gm1 * inv
  sm2 = gm2 * inv

  wA = Wm1[:, :h]
  wB = Wm1[:, h:2 * h]
  wE = Wm1[:, 2 * h:]
  wC = wE @ W_edge
  c0 = (bm1 + wE @ b_edge).reshape(1, h)
  w2p = Wm2 * sm1[None, :]
  b2p = (bm2 + Wm2 @ btm1).reshape(1, h // 2)
  w3p = Wm3 * sm2[None, :]
  b3p = (bm3 + Wm3 @ btm2).reshape(1, Wm3.shape[0])

  ce1 = (1.0 + eps1).reshape(1, 1)
  ce2 = (1.0 + eps2).reshape(1, 1)

  h0s = _node_proj_call(x, W_node.T, b_node.reshape(1, h))

  agg1 = _segment_sum(h0s.reshape(2 * n, 128), src_p, dst_p, n)

  h1s = _gin_mlp_call(h0s, agg1.reshape(2, n, 128), ce1,
                      W1a.T, b1a.reshape(1, h), W1b.T, b1b.reshape(1, h),
                      s1, bt1.reshape(1, h), split_out=True)

  agg2 = _segment_sum(h1s.reshape(2 * n, 128), src_p, dst_p, n)

  waT = wA.T
  wbT = wB.T
  a_tbl, b_tbl = _gin2_ab_call(h1s, agg2.reshape(2, n, 128), ce2,
                               W2a.T, b2a.reshape(1, h),
                               W2b.T, b2b.reshape(1, h),
                               s2, bt2.reshape(1, h),
                               waT[:, 0::2], waT[:, 1::2],
                               c0[:, 0::2], c0[:, 1::2],
                               wbT[:, 0::2], wbT[:, 1::2])

  wcT = wC.T
  wcT_p = jnp.concatenate([wcT[:, 0::2], wcT[:, 1::2]], axis=1)
  w2T = w2p.T
  w2T_p = jnp.concatenate([w2T[0::2], w2T[1::2]], axis=0)

  n_chunks = 4
  ch = n_edges // n_chunks
  gquant = _EB * 2 * _NC * _NS
  ch_pad = (-ch) % gquant
  zc_i = jnp.zeros((ch_pad,), jnp.int32)
  zc_f = jnp.zeros((ch_pad, edge_attr.shape[1]), jnp.float32)
  outs = []
  for i in range(n_chunks):
    sl = slice(i * ch, (i + 1) * ch)
    ga, gb = _gather_pair(a_tbl, b_tbl,
                          jnp.concatenate([src[sl], zc_i]),
                          jnp.concatenate([dst[sl], zc_i]))
    ea_c = jnp.concatenate([edge_attr[sl], zc_f], axis=0)
    outs.append(_edge_head_call(ga, gb, ea_c, wcT_p, w2T_p,
                                b2p, w3p.T, b3p)[:ch])
  return jnp.concatenate(outs, axis=0)


# speedup vs baseline: 1.4218x; 1.4218x over previous
import functools

import jax
import jax.numpy as jnp
from jax import lax
from jax.experimental import pallas as pl
from jax.experimental.pallas import tpu as pltpu
from jax.experimental.pallas import tpu_sc as plsc

_BN_EPS = 1e-5

_NC = 2
_NS = 16
_L = 16
_EB = 128


def _segment_sum(tbl2, src_p, dst_p, n_nodes, d=128):
  n_edges = src_p.shape[0]
  nb = n_edges // _EB
  nit = nb // _NS
  zrows = 80
  nblk = n_nodes // zrows

  mesh = plsc.VectorSubcoreMesh(core_axis_name="c", subcore_axis_name="s")

  @functools.partial(
      pl.kernel,
      out_type=jax.ShapeDtypeStruct((2 * n_nodes, d), jnp.float32),
      name="segsum_fsplit",
      mesh=mesh,
      scratch_types=[
          pltpu.VMEM((_EB,), jnp.int32),
          pltpu.VMEM((_EB,), jnp.int32),
          pltpu.VMEM((_EB,), jnp.int32),
          pltpu.VMEM((_EB, d), jnp.float32),
          pltpu.VMEM((_EB,), jnp.int32),
          pltpu.VMEM((_EB,), jnp.int32),
          pltpu.VMEM((_EB,), jnp.int32),
          pltpu.VMEM((_EB, d), jnp.float32),
          pltpu.VMEM((zrows, d), jnp.float32),
          pltpu.VMEM_SHARED((n_nodes + 8, d), jnp.float32),
          pltpu.SemaphoreType.DMA,
          pltpu.SemaphoreType.DMA,
          pltpu.SemaphoreType.DMA,
          pltpu.SemaphoreType.DMA,
      ],
  )
  def seg_kernel(tbl_ref, src_ref, dst_ref, out_ref,
                 src_v0, dst_v0, adj_v0, rows_v0,
                 src_v1, dst_v1, adj_v1, rows_v1,
                 zbuf, acc, sg0, sg1, ss0, ss1):
    c = lax.axis_index("c")
    s = lax.axis_index("s")
    zero16 = jnp.zeros((_L,), jnp.float32)
    src_vs = (src_v0, src_v1)
    dst_vs = (dst_v0, dst_v1)
    adj_vs = (adj_v0, adj_v1)
    rows_vs = (rows_v0, rows_v1)
    sgs = (sg0, sg1)
    sss = (ss0, ss1)

    @pl.loop(0, zrows)
    def _zero_zbuf(i):
      for j in range(d // _L):
        zbuf[i, pl.ds(j * _L, _L)] = zero16

    @pl.loop(s, nblk, step=_NS)
    def _zero_acc(k):
      pltpu.sync_copy(zbuf, acc.at[pl.ds(k * zrows, zrows)])

    plsc.subcore_barrier()

    off16 = jnp.full((_L,), c * n_nodes, jnp.int32)

    def fire_gather(slot, b):
      base = b * _EB
      pltpu.sync_copy(src_ref.at[pl.ds(base, _EB)], src_vs[slot])
      pltpu.sync_copy(dst_ref.at[pl.ds(base, _EB)], dst_vs[slot])
      for j in range(_EB // _L):
        sl = pl.ds(j * _L, _L)
        adj_vs[slot][sl] = src_vs[slot][sl] + off16
      pltpu.async_copy(tbl_ref.at[adj_vs[slot]], rows_vs[slot], sgs[slot])

    def wait_gather(slot):
      pltpu.make_async_copy(tbl_ref.at[pl.ds(0, _EB)],
                            rows_vs[slot], sgs[slot]).wait()

    def fire_scatter(slot):
      pltpu.async_copy(rows_vs[slot], acc.at[dst_vs[slot]], sss[slot],
                       add=True)

    def wait_scatter(slot):
      pltpu.make_async_copy(tbl_ref.at[pl.ds(0, _EB)],
                            rows_vs[slot], sss[slot]).wait()

    fire_gather(0, s)
    fire_gather(1, s + _NS)

    @pl.loop(0, (nit - 2) // 2)
    def _pairs(p):
      b0 = s + (2 * p) * _NS
      wait_gather(0)
      fire_scatter(0)
      wait_gather(1)
      fire_scatter(1)
      wait_scatter(0)
      fire_gather(0, b0 + 2 * _NS)
      wait_scatter(1)
      fire_gather(1, b0 + 3 * _NS)

    wait_gather(0)
    fire_scatter(0)
    wait_gather(1)
    fire_scatter(1)
    wait_scatter(0)
    wait_scatter(1)

    plsc.subcore_barrier()

    @pl.loop(s, nblk, step=_NS)
    def _writeback(k):
      r0 = k * zrows
      pltpu.sync_copy(acc.at[pl.ds(r0, zrows)],
                      out_ref.at[pl.ds(c * n_nodes + r0, zrows)])

  return seg_kernel(tbl2, src_p, dst_p)


def _gather_pair(a_tbl, b_tbl, src, dst):
  n_edges = src.shape[0]
  nb = n_edges // _EB
  d = a_tbl.shape[1]

  mesh = plsc.VectorSubcoreMesh(core_axis_name="c", subcore_axis_name="s")

  @functools.partial(
      pl.kernel,
      out_type=[
          jax.ShapeDtypeStruct((n_edges, d), jnp.uint32),
          jax.ShapeDtypeStruct((n_edges, d), jnp.uint32),
      ],
      name="gather_pair",
      mesh=mesh,
      scratch_types=[
          pltpu.VMEM((_EB,), jnp.int32),
          pltpu.VMEM((_EB,), jnp.int32),
          pltpu.VMEM((_EB, 128), jnp.uint32),
          pltpu.VMEM((_EB, 128), jnp.uint32),
          pltpu.SemaphoreType.DMA,
          pltpu.SemaphoreType.DMA,
      ],
  )
  def ga_kernel(a_ref, b_ref, src_ref, dst_ref, outa_ref, outb_ref,
                src_v, dst_v, arows, brows, sem_a, sem_b):
    c = lax.axis_index("c")
    s = lax.axis_index("s")
    wid = s * _NC + c

    @pl.loop(wid, nb, step=_NC * _NS)
    def _batch(b):
      base = b * _EB
      pltpu.sync_copy(src_ref.at[pl.ds(base, _EB)], src_v)
      pltpu.sync_copy(dst_ref.at[pl.ds(base, _EB)], dst_v)
      cp_a = pltpu.async_copy(a_ref.at[src_v], arows, sem_a)
      cp_b = pltpu.async_copy(b_ref.at[dst_v], brows, sem_b)
      cp_a.wait()
      cp_b.wait()
      pltpu.sync_copy(arows, outa_ref.at[pl.ds(base, _EB)])
      pltpu.sync_copy(brows, outb_ref.at[pl.ds(base, _EB)])

  return ga_kernel(a_tbl, b_tbl, src, dst)


def _full2d(r, c):
  return pl.BlockSpec((r, c), lambda i: (0, 0))


def _node_proj_call(x, wnT, bn):
  n, d_in = x.shape
  h = wnT.shape[1]
  bn_rows = 1000
  grid = n // bn_rows

  def body(x_ref, w_ref, b_ref, o_ref):
    h0 = jnp.dot(x_ref[...], w_ref[...],
                 preferred_element_type=jnp.float32) + b_ref[...]
    o_ref[0] = h0[:, :128]
    o_ref[1] = h0[:, 128:]

  return pl.pallas_call(
      body,
      grid=(grid,),
      in_specs=[
          pl.BlockSpec((bn_rows, d_in), lambda i: (i, 0)),
          _full2d(d_in, h),
          _full2d(1, h),
      ],
      out_specs=pl.BlockSpec((2, bn_rows, 128), lambda i: (0, i, 0)),
      out_shape=jax.ShapeDtypeStruct((2, n, 128), jnp.float32),
  )(x, wnT, bn)


def _gin_mlp_call(h2s, agg2s, ce, waT, ba, wbT, bb, scale, shift,
                  split_out):
  n = h2s.shape[1]
  h = 256
  bn_rows = 1000
  grid = n // bn_rows

  def body(h_ref, a_ref, ce_ref, wa_ref, ba_ref, wb_ref, bb_ref,
           sc_ref, sh_ref, o_ref):
    hcat = jnp.concatenate([h_ref[0], h_ref[1]], axis=1)
    acat = jnp.concatenate([a_ref[0], a_ref[1]], axis=1)
    t = ce_ref[0, 0] * hcat + acat
    t = jnp.maximum(jnp.dot(t, wa_ref[...],
                            preferred_element_type=jnp.float32) + ba_ref[...],
                    0.0)
    t = jnp.maximum(jnp.dot(t, wb_ref[...],
                            preferred_element_type=jnp.float32) + bb_ref[...],
                    0.0)
    t = jnp.maximum(t * sc_ref[...] + sh_ref[...], 0.0)
    if split_out:
      o_ref[0] = t[:, :128]
      o_ref[1] = t[:, 128:]
    else:
      o_ref[...] = t

  if split_out:
    out_spec = pl.BlockSpec((2, bn_rows, 128), lambda i: (0, i, 0))
    out_shape = jax.ShapeDtypeStruct((2, n, 128), jnp.float32)
  else:
    out_spec = pl.BlockSpec((bn_rows, h), lambda i: (i, 0))
    out_shape = jax.ShapeDtypeStruct((n, h), jnp.float32)

  return pl.pallas_call(
      body,
      grid=(grid,),
      in_specs=[
          pl.BlockSpec((2, bn_rows, 128), lambda i: (0, i, 0)),
          pl.BlockSpec((2, bn_rows, 128), lambda i: (0, i, 0)),
          _full2d(1, 1),
          _full2d(h, h),
          _full2d(1, h),
          _full2d(h, h),
          _full2d(1, h),
          _full2d(1, h),
          _full2d(1, h),
      ],
      out_specs=out_spec,
      out_shape=out_shape,
  )(h2s, agg2s, ce, waT, ba, wbT, bb, scale, shift)


def _round_pack_bf16(even, odd):
  ue = lax.bitcast_convert_type(even, jnp.uint32)
  uo = lax.bitcast_convert_type(odd, jnp.uint32)
  ue = ue + jnp.uint32(0x7FFF) + ((ue >> 16) & jnp.uint32(1))
  uo = uo + jnp.uint32(0x7FFF) + ((uo >> 16) & jnp.uint32(1))
  return (ue >> 16) | (uo & jnp.uint32(0xFFFF0000))


def _unpack_bf16(word):
  f_even = lax.bitcast_convert_type(word << 16, jnp.float32)
  f_odd = lax.bitcast_convert_type(word & jnp.uint32(0xFFFF0000), jnp.float32)
  return f_even, f_odd


def _gin2_ab_call(h1s, agg2s, ce, waT, ba, wbT, bb, scale, shift,
                  wAT_e, wAT_o, ca_e, ca_o, wBT_e, wBT_o):
  n = h1s.shape[1]
  h = 256
  hw = h // 2
  bn_rows = 1000
  grid = n // bn_rows

  def body(h_ref, a_ref, ce_ref, wa_ref, ba_ref, wb_ref, bb_ref,
           sc_ref, sh_ref, wae_ref, wao_ref, cae_ref, cao_ref,
           wbe_ref, wbo_ref, oa_ref, ob_ref):
    hcat = jnp.concatenate([h_ref[0], h_ref[1]], axis=1)
    acat = jnp.concatenate([a_ref[0], a_ref[1]], axis=1)
    t = ce_ref[0, 0] * hcat + acat
    t = jnp.maximum(jnp.dot(t, wa_ref[...],
                            preferred_element_type=jnp.float32) + ba_ref[...],
                    0.0)
    t = jnp.maximum(jnp.dot(t, wb_ref[...],
                            preferred_element_type=jnp.float32) + bb_ref[...],
                    0.0)
    h2 = jnp.maximum(t * sc_ref[...] + sh_ref[...], 0.0)
    ae = jnp.dot(h2, wae_ref[...], preferred_element_type=jnp.float32) \
        + cae_ref[...]
    ao = jnp.dot(h2, wao_ref[...], preferred_element_type=jnp.float32) \
        + cao_ref[...]
    be = jnp.dot(h2, wbe_ref[...], preferred_element_type=jnp.float32)
    bo = jnp.dot(h2, wbo_ref[...], preferred_element_type=jnp.float32)
    oa_ref[...] = _round_pack_bf16(ae, ao)
    ob_ref[...] = _round_pack_bf16(be, bo)

  return pl.pallas_call(
      body,
      grid=(grid,),
      in_specs=[
          pl.BlockSpec((2, bn_rows, 128), lambda i: (0, i, 0)),
          pl.BlockSpec((2, bn_rows, 128), lambda i: (0, i, 0)),
          _full2d(1, 1),
          _full2d(h, h),
          _full2d(1, h),
          _full2d(h, h),
          _full2d(1, h),
          _full2d(1, h),
          _full2d(1, h),
          _full2d(h, hw),
          _full2d(h, hw),
          _full2d(1, hw),
          _full2d(1, hw),
          _full2d(h, hw),
          _full2d(h, hw),
      ],
      out_specs=[
          pl.BlockSpec((bn_rows, hw), lambda i: (i, 0)),
          pl.BlockSpec((bn_rows, hw), lambda i: (i, 0)),
      ],
      out_shape=[
          jax.ShapeDtypeStruct((n, hw), jnp.uint32),
          jax.ShapeDtypeStruct((n, hw), jnp.uint32),
      ],
  )(h1s, agg2s, ce, waT, ba, wbT, bb, scale, shift,
    wAT_e, wAT_o, ca_e, ca_o, wBT_e, wBT_o)


def _edge_head_call(ga, gb, edge_attr, wcT_p, w2T_p, b2, w3T, b3):
  e = ga.shape[0]
  be_rows = 2000
  grid = e // be_rows
  d_edge = edge_attr.shape[1]
  h = 2 * ga.shape[1]
  h2 = w2T_p.shape[1]
  c_out = w3T.shape[1]

  def body(ga_ref, gb_ref, ea_ref, wc_ref, w2_ref, b2_ref, w3_ref, b3_ref,
           o_ref):
    ec = jnp.dot(ea_ref[...], wc_ref[...], preferred_element_type=jnp.float32)
    a_even, a_odd = _unpack_bf16(ga_ref[...])
    b_even, b_odd = _unpack_bf16(gb_ref[...])
    gp = jnp.concatenate([a_even + b_even, a_odd + b_odd], axis=1)
    z1 = jnp.maximum(gp + ec, 0.0)
    z2 = jnp.maximum(jnp.dot(z1, w2_ref[...],
                             preferred_element_type=jnp.float32) + b2_ref[...],
                     0.0)
    o_ref[...] = jnp.dot(z2, w3_ref[...],
                         preferred_element_type=jnp.float32) + b3_ref[...]

  return pl.pallas_call(
      body,
      grid=(grid,),
      in_specs=[
          pl.BlockSpec((be_rows, h // 2), lambda i: (i, 0)),
          pl.BlockSpec((be_rows, h // 2), lambda i: (i, 0)),
          pl.BlockSpec((be_rows, d_edge), lambda i: (i, 0)),
          _full2d(d_edge, h),
          _full2d(h, h2),
          _full2d(1, h2),
          _full2d(h2, c_out),
          _full2d(1, c_out),
      ],
      out_specs=pl.BlockSpec((be_rows, c_out), lambda i: (i, 0)),
      out_shape=jax.ShapeDtypeStruct((e, c_out), jnp.float32),
  )(ga, gb, edge_attr, wcT_p, w2T_p, b2, w3T, b3)


def kernel(x, edge_index, edge_attr, W_node, b_node, W_edge, b_edge,
           eps1, W1a, b1a, W1b, b1b, g1, bt1,
           eps2, W2a, b2a, W2b, b2b, g2, bt2,
           Wm1, bm1, gm1, btm1, Wm2, bm2, gm2, btm2, Wm3, bm3):
  n = x.shape[0]
  h = W_node.shape[0]

  src = edge_index[0]
  dst = edge_index[1]

  n_edges = src.shape[0]
  quant = _EB * 2 * _NS
  n_pad = (-n_edges) % quant
  if (n_edges + n_pad) // _EB // _NS < 4:
    n_pad += 2 * quant
  src_p = jnp.concatenate([src, jnp.zeros((n_pad,), jnp.int32)])
  dst_p = jnp.concatenate([dst, jnp.full((n_pad,), n, jnp.int32)])

  inv = 1.0 / jnp.sqrt(jnp.float32(1.0 + _BN_EPS))
  s1 = (g1 * inv).reshape(1, h)
  s2 = (g2 * inv).reshape(1, h)
  sm1 = gm1 * inv
  sm2 = gm2 * inv

  wA = Wm1[:, :h]
  wB = Wm1[:, h:2 * h]
  wE = Wm1[:, 2 * h:]
  wC = wE @ W_edge
  c0 = (bm1 + wE @ b_edge).reshape(1, h)
  w2p = Wm2 * sm1[None, :]
  b2p = (bm2 + Wm2 @ btm1).reshape(1, h // 2)
  w3p = Wm3 * sm2[None, :]
  b3p = (bm3 + Wm3 @ btm2).reshape(1, Wm3.shape[0])

  ce1 = (1.0 + eps1).reshape(1, 1)
  ce2 = (1.0 + eps2).reshape(1, 1)

  h0s = _node_proj_call(x, W_node.T, b_node.reshape(1, h))

  agg1 = _segment_sum(h0s.reshape(2 * n, 128), src_p, dst_p, n)

  h1s = _gin_mlp_call(h0s, agg1.reshape(2, n, 128), ce1,
                      W1a.T, b1a.reshape(1, h), W1b.T, b1b.reshape(1, h),
                      s1, bt1.reshape(1, h), split_out=True)

  agg2 = _segment_sum(h1s.reshape(2 * n, 128), src_p, dst_p, n)

  waT = wA.T
  wbT = wB.T
  a_tbl, b_tbl = _gin2_ab_call(h1s, agg2.reshape(2, n, 128), ce2,
                               W2a.T, b2a.reshape(1, h),
                               W2b.T, b2b.reshape(1, h),
                               s2, bt2.reshape(1, h),
                               waT[:, 0::2], waT[:, 1::2],
                               c0[:, 0::2], c0[:, 1::2],
                               wbT[:, 0::2], wbT[:, 1::2])

  wcT = wC.T
  wcT_p = jnp.concatenate([wcT[:, 0::2], wcT[:, 1::2]], axis=1)
  w2T = w2p.T
  w2T_p = jnp.concatenate([w2T[0::2], w2T[1::2]], axis=0)

  n_chunks = 4
  ch = n_edges // n_chunks
  outs = []
  for i in range(n_chunks):
    sl = slice(i * ch, (i + 1) * ch)
    ga, gb = _gather_pair(a_tbl, b_tbl, src[sl], dst[sl])
    outs.append(_edge_head_call(ga, gb, edge_attr[sl], wcT_p, w2T_p,
                                b2p, w3p.T, b3p))
  return jnp.concatenate(outs, axis=0)


# speedup vs baseline: 1.4265x; 1.0033x over previous
import functools

import jax
import jax.numpy as jnp
from jax import lax
from jax.experimental import pallas as pl
from jax.experimental.pallas import tpu as pltpu
from jax.experimental.pallas import tpu_sc as plsc

_BN_EPS = 1e-5

_NC = 2
_NS = 16
_L = 16
_EB = 128


def _segment_sum(tbl2, src_p, dst_p, n_nodes, d=128):
  n_edges = src_p.shape[0]
  nb = n_edges // _EB
  nit = nb // _NS
  zrows = 80
  nblk = n_nodes // zrows

  mesh = plsc.VectorSubcoreMesh(core_axis_name="c", subcore_axis_name="s")

  @functools.partial(
      pl.kernel,
      out_type=jax.ShapeDtypeStruct((2 * n_nodes, d), jnp.float32),
      name="segsum_fsplit",
      mesh=mesh,
      scratch_types=[
          pltpu.VMEM((_EB,), jnp.int32),
          pltpu.VMEM((_EB,), jnp.int32),
          pltpu.VMEM((_EB,), jnp.int32),
          pltpu.VMEM((_EB, d), jnp.float32),
          pltpu.VMEM((_EB,), jnp.int32),
          pltpu.VMEM((_EB,), jnp.int32),
          pltpu.VMEM((_EB,), jnp.int32),
          pltpu.VMEM((_EB, d), jnp.float32),
          pltpu.VMEM((zrows, d), jnp.float32),
          pltpu.VMEM_SHARED((n_nodes + 8, d), jnp.float32),
          pltpu.SemaphoreType.DMA,
          pltpu.SemaphoreType.DMA,
          pltpu.SemaphoreType.DMA,
          pltpu.SemaphoreType.DMA,
      ],
  )
  def seg_kernel(tbl_ref, src_ref, dst_ref, out_ref,
                 src_v0, dst_v0, adj_v0, rows_v0,
                 src_v1, dst_v1, adj_v1, rows_v1,
                 zbuf, acc, sg0, sg1, ss0, ss1):
    c = lax.axis_index("c")
    s = lax.axis_index("s")
    zero16 = jnp.zeros((_L,), jnp.float32)
    src_vs = (src_v0, src_v1)
    dst_vs = (dst_v0, dst_v1)
    adj_vs = (adj_v0, adj_v1)
    rows_vs = (rows_v0, rows_v1)
    sgs = (sg0, sg1)
    sss = (ss0, ss1)

    @pl.loop(0, zrows)
    def _zero_zbuf(i):
      for j in range(d // _L):
        zbuf[i, pl.ds(j * _L, _L)] = zero16

    @pl.loop(s, nblk, step=_NS)
    def _zero_acc(k):
      pltpu.sync_copy(zbuf, acc.at[pl.ds(k * zrows, zrows)])

    plsc.subcore_barrier()

    off16 = jnp.full((_L,), c * n_nodes, jnp.int32)

    def fire_gather(slot, b):
      base = b * _EB
      pltpu.sync_copy(src_ref.at[pl.ds(base, _EB)], src_vs[slot])
      pltpu.sync_copy(dst_ref.at[pl.ds(base, _EB)], dst_vs[slot])
      for j in range(_EB // _L):
        sl = pl.ds(j * _L, _L)
        adj_vs[slot][sl] = src_vs[slot][sl] + off16
      pltpu.async_copy(tbl_ref.at[adj_vs[slot]], rows_vs[slot], sgs[slot])

    def wait_gather(slot):
      pltpu.make_async_copy(tbl_ref.at[pl.ds(0, _EB)],
                            rows_vs[slot], sgs[slot]).wait()

    def fire_scatter(slot):
      pltpu.async_copy(rows_vs[slot], acc.at[dst_vs[slot]], sss[slot],
                       add=True)

    def wait_scatter(slot):
      pltpu.make_async_copy(tbl_ref.at[pl.ds(0, _EB)],
                            rows_vs[slot], sss[slot]).wait()

    fire_gather(0, s)
    fire_gather(1, s + _NS)

    @pl.loop(0, (nit - 2) // 2)
    def _pairs(p):
      b0 = s + (2 * p) * _NS
      wait_gather(0)
      fire_scatter(0)
      wait_gather(1)
      fire_scatter(1)
      wait_scatter(0)
      fire_gather(0, b0 + 2 * _NS)
      wait_scatter(1)
      fire_gather(1, b0 + 3 * _NS)

    wait_gather(0)
    fire_scatter(0)
    wait_gather(1)
    fire_scatter(1)
    wait_scatter(0)
    wait_scatter(1)

    plsc.subcore_barrier()

    @pl.loop(s, nblk, step=_NS)
    def _writeback(k):
      r0 = k * zrows
      pltpu.sync_copy(acc.at[pl.ds(r0, zrows)],
                      out_ref.at[pl.ds(c * n_nodes + r0, zrows)])

  return seg_kernel(tbl2, src_p, dst_p)


def _gather_pair(a_tbl, b_tbl, src, dst):
  n_edges = src.shape[0]
  nb = n_edges // _EB
  d = a_tbl.shape[1]

  mesh = plsc.VectorSubcoreMesh(core_axis_name="c", subcore_axis_name="s")

  @functools.partial(
      pl.kernel,
      out_type=[
          jax.ShapeDtypeStruct((n_edges, d), jnp.uint32),
          jax.ShapeDtypeStruct((n_edges, d), jnp.uint32),
      ],
      name="gather_pair",
      mesh=mesh,
      scratch_types=[
          pltpu.VMEM((_EB,), jnp.int32),
          pltpu.VMEM((_EB,), jnp.int32),
          pltpu.VMEM((_EB, 128), jnp.uint32),
          pltpu.VMEM((_EB, 128), jnp.uint32),
          pltpu.SemaphoreType.DMA,
          pltpu.SemaphoreType.DMA,
      ],
  )
  def ga_kernel(a_ref, b_ref, src_ref, dst_ref, outa_ref, outb_ref,
                src_v, dst_v, arows, brows, sem_a, sem_b):
    c = lax.axis_index("c")
    s = lax.axis_index("s")
    wid = s * _NC + c

    @pl.loop(wid, nb, step=_NC * _NS)
    def _batch(b):
      base = b * _EB
      pltpu.sync_copy(src_ref.at[pl.ds(base, _EB)], src_v)
      pltpu.sync_copy(dst_ref.at[pl.ds(base, _EB)], dst_v)
      cp_a = pltpu.async_copy(a_ref.at[src_v], arows, sem_a)
      cp_b = pltpu.async_copy(b_ref.at[dst_v], brows, sem_b)
      cp_a.wait()
      cp_b.wait()
      pltpu.sync_copy(arows, outa_ref.at[pl.ds(base, _EB)])
      pltpu.sync_copy(brows, outb_ref.at[pl.ds(base, _EB)])

  return ga_kernel(a_tbl, b_tbl, src, dst)


def _full2d(r, c):
  return pl.BlockSpec((r, c), lambda i: (0, 0))


def _node_proj_call(x, wnT, bn):
  n, d_in = x.shape
  h = wnT.shape[1]
  bn_rows = 1000
  grid = n // bn_rows

  def body(x_ref, w_ref, b_ref, o_ref):
    h0 = jnp.dot(x_ref[...], w_ref[...],
                 preferred_element_type=jnp.float32) + b_ref[...]
    o_ref[0] = h0[:, :128]
    o_ref[1] = h0[:, 128:]

  return pl.pallas_call(
      body,
      grid=(grid,),
      in_specs=[
          pl.BlockSpec((bn_rows, d_in), lambda i: (i, 0)),
          _full2d(d_in, h),
          _full2d(1, h),
      ],
      out_specs=pl.BlockSpec((2, bn_rows, 128), lambda i: (0, i, 0)),
      out_shape=jax.ShapeDtypeStruct((2, n, 128), jnp.float32),
  )(x, wnT, bn)


def _gin_mlp_call(h2s, agg2s, ce, waT, ba, wbT, bb, scale, shift,
                  split_out):
  n = h2s.shape[1]
  h = 256
  bn_rows = 1000
  grid = n // bn_rows

  def body(h_ref, a_ref, ce_ref, wa_ref, ba_ref, wb_ref, bb_ref,
           sc_ref, sh_ref, o_ref):
    hcat = jnp.concatenate([h_ref[0], h_ref[1]], axis=1)
    acat = jnp.concatenate([a_ref[0], a_ref[1]], axis=1)
    t = ce_ref[0, 0] * hcat + acat
    t = jnp.maximum(jnp.dot(t, wa_ref[...],
                            preferred_element_type=jnp.float32) + ba_ref[...],
                    0.0)
    t = jnp.maximum(jnp.dot(t, wb_ref[...],
                            preferred_element_type=jnp.float32) + bb_ref[...],
                    0.0)
    t = jnp.maximum(t * sc_ref[...] + sh_ref[...], 0.0)
    if split_out:
      o_ref[0] = t[:, :128]
      o_ref[1] = t[:, 128:]
    else:
      o_ref[...] = t

  if split_out:
    out_spec = pl.BlockSpec((2, bn_rows, 128), lambda i: (0, i, 0))
    out_shape = jax.ShapeDtypeStruct((2, n, 128), jnp.float32)
  else:
    out_spec = pl.BlockSpec((bn_rows, h), lambda i: (i, 0))
    out_shape = jax.ShapeDtypeStruct((n, h), jnp.float32)

  return pl.pallas_call(
      body,
      grid=(grid,),
      in_specs=[
          pl.BlockSpec((2, bn_rows, 128), lambda i: (0, i, 0)),
          pl.BlockSpec((2, bn_rows, 128), lambda i: (0, i, 0)),
          _full2d(1, 1),
          _full2d(h, h),
          _full2d(1, h),
          _full2d(h, h),
          _full2d(1, h),
          _full2d(1, h),
          _full2d(1, h),
      ],
      out_specs=out_spec,
      out_shape=out_shape,
  )(h2s, agg2s, ce, waT, ba, wbT, bb, scale, shift)


def _round_pack_bf16(even, odd):
  ue = lax.bitcast_convert_type(even, jnp.uint32)
  uo = lax.bitcast_convert_type(odd, jnp.uint32)
  ue = ue + jnp.uint32(0x7FFF) + ((ue >> 16) & jnp.uint32(1))
  uo = uo + jnp.uint32(0x7FFF) + ((uo >> 16) & jnp.uint32(1))
  return (ue >> 16) | (uo & jnp.uint32(0xFFFF0000))


def _unpack_bf16(word):
  f_even = lax.bitcast_convert_type(word << 16, jnp.float32)
  f_odd = lax.bitcast_convert_type(word & jnp.uint32(0xFFFF0000), jnp.float32)
  return f_even, f_odd


def _gin2_ab_call(h1s, agg2s, ce, waT, ba, wbT, bb, scale, shift,
                  wAT_e, wAT_o, ca_e, ca_o, wBT_e, wBT_o):
  n = h1s.shape[1]
  h = 256
  hw = h // 2
  bn_rows = 1000
  grid = n // bn_rows

  def body(h_ref, a_ref, ce_ref, wa_ref, ba_ref, wb_ref, bb_ref,
           sc_ref, sh_ref, wae_ref, wao_ref, cae_ref, cao_ref,
           wbe_ref, wbo_ref, oa_ref, ob_ref):
    hcat = jnp.concatenate([h_ref[0], h_ref[1]], axis=1)
    acat = jnp.concatenate([a_ref[0], a_ref[1]], axis=1)
    t = ce_ref[0, 0] * hcat + acat
    t = jnp.maximum(jnp.dot(t, wa_ref[...],
                            preferred_element_type=jnp.float32) + ba_ref[...],
                    0.0)
    t = jnp.maximum(jnp.dot(t, wb_ref[...],
                            preferred_element_type=jnp.float32) + bb_ref[...],
                    0.0)
    h2 = jnp.maximum(t * sc_ref[...] + sh_ref[...], 0.0)
    ae = jnp.dot(h2, wae_ref[...], preferred_element_type=jnp.float32) \
        + cae_ref[...]
    ao = jnp.dot(h2, wao_ref[...], preferred_element_type=jnp.float32) \
        + cao_ref[...]
    be = jnp.dot(h2, wbe_ref[...], preferred_element_type=jnp.float32)
    bo = jnp.dot(h2, wbo_ref[...], preferred_element_type=jnp.float32)
    oa_ref[...] = _round_pack_bf16(ae, ao)
    ob_ref[...] = _round_pack_bf16(be, bo)

  return pl.pallas_call(
      body,
      grid=(grid,),
      in_specs=[
          pl.BlockSpec((2, bn_rows, 128), lambda i: (0, i, 0)),
          pl.BlockSpec((2, bn_rows, 128), lambda i: (0, i, 0)),
          _full2d(1, 1),
          _full2d(h, h),
          _full2d(1, h),
          _full2d(h, h),
          _full2d(1, h),
          _full2d(1, h),
          _full2d(1, h),
          _full2d(h, hw),
          _full2d(h, hw),
          _full2d(1, hw),
          _full2d(1, hw),
          _full2d(h, hw),
          _full2d(h, hw),
      ],
      out_specs=[
          pl.BlockSpec((bn_rows, hw), lambda i: (i, 0)),
          pl.BlockSpec((bn_rows, hw), lambda i: (i, 0)),
      ],
      out_shape=[
          jax.ShapeDtypeStruct((n, hw), jnp.uint32),
          jax.ShapeDtypeStruct((n, hw), jnp.uint32),
      ],
  )(h1s, agg2s, ce, waT, ba, wbT, bb, scale, shift,
    wAT_e, wAT_o, ca_e, ca_o, wBT_e, wBT_o)


def _edge_head_call(ga, gb, edge_attr, wcT_p, w2T_p, b2, w3T, b3):
  e = ga.shape[0]
  be_rows = 2000
  grid = e // be_rows
  d_edge = edge_attr.shape[1]
  h = 2 * ga.shape[1]
  h2 = w2T_p.shape[1]
  c_out = w3T.shape[1]

  def body(ga_ref, gb_ref, ea_ref, wc_ref, w2_ref, b2_ref, w3_ref, b3_ref,
           o_ref):
    ec = jnp.dot(ea_ref[...], wc_ref[...], preferred_element_type=jnp.float32)
    a_even, a_odd = _unpack_bf16(ga_ref[...])
    b_even, b_odd = _unpack_bf16(gb_ref[...])
    gp = jnp.concatenate([a_even + b_even, a_odd + b_odd], axis=1)
    z1 = jnp.maximum(gp + ec, 0.0)
    z2 = jnp.maximum(jnp.dot(z1, w2_ref[...],
                             preferred_element_type=jnp.float32) + b2_ref[...],
                     0.0)
    o_ref[...] = jnp.dot(z2, w3_ref[...],
                         preferred_element_type=jnp.float32) + b3_ref[...]

  return pl.pallas_call(
      body,
      grid=(grid,),
      in_specs=[
          pl.BlockSpec((be_rows, h // 2), lambda i: (i, 0)),
          pl.BlockSpec((be_rows, h // 2), lambda i: (i, 0)),
          pl.BlockSpec((be_rows, d_edge), lambda i: (i, 0)),
          _full2d(d_edge, h),
          _full2d(h, h2),
          _full2d(1, h2),
          _full2d(h2, c_out),
          _full2d(1, c_out),
      ],
      out_specs=pl.BlockSpec((be_rows, c_out), lambda i: (i, 0)),
      out_shape=jax.ShapeDtypeStruct((e, c_out), jnp.float32),
  )(ga, gb, edge_attr, wcT_p, w2T_p, b2, w3T, b3)


def kernel(x, edge_index, edge_attr, W_node, b_node, W_edge, b_edge,
           eps1, W1a, b1a, W1b, b1b, g1, bt1,
           eps2, W2a, b2a, W2b, b2b, g2, bt2,
           Wm1, bm1, gm1, btm1, Wm2, bm2, gm2, btm2, Wm3, bm3):
  n = x.shape[0]
  h = W_node.shape[0]

  src = edge_index[0]
  dst = edge_index[1]

  n_edges = src.shape[0]
  quant = _EB * 2 * _NS
  n_pad = (-n_edges) % quant
  if (n_edges + n_pad) // _EB // _NS < 4:
    n_pad += 2 * quant
  src_p = jnp.concatenate([src, jnp.zeros((n_pad,), jnp.int32)])
  dst_p = jnp.concatenate([dst, jnp.full((n_pad,), n, jnp.int32)])

  inv = 1.0 / jnp.sqrt(jnp.float32(1.0 + _BN_EPS))
  s1 = (g1 * inv).reshape(1, h)
  s2 = (g2 * inv).reshape(1, h)
  sm1 = gm1 * inv
  sm2 = gm2 * inv

  wA = Wm1[:, :h]
  wB = Wm1[:, h:2 * h]
  wE = Wm1[:, 2 * h:]
  wC = wE @ W_edge
  c0 = (bm1 + wE @ b_edge).reshape(1, h)
  w2p = Wm2 * sm1[None, :]
  b2p = (bm2 + Wm2 @ btm1).reshape(1, h // 2)
  w3p = Wm3 * sm2[None, :]
  b3p = (bm3 + Wm3 @ btm2).reshape(1, Wm3.shape[0])

  ce1 = (1.0 + eps1).reshape(1, 1)
  ce2 = (1.0 + eps2).reshape(1, 1)

  h0s = _node_proj_call(x, W_node.T, b_node.reshape(1, h))

  agg1 = _segment_sum(h0s.reshape(2 * n, 128), src_p, dst_p, n)

  h1s = _gin_mlp_call(h0s, agg1.reshape(2, n, 128), ce1,
                      W1a.T, b1a.reshape(1, h), W1b.T, b1b.reshape(1, h),
                      s1, bt1.reshape(1, h), split_out=True)

  agg2 = _segment_sum(h1s.reshape(2 * n, 128), src_p, dst_p, n)

  waT = wA.T
  wbT = wB.T
  a_tbl, b_tbl = _gin2_ab_call(h1s, agg2.reshape(2, n, 128), ce2,
                               W2a.T, b2a.reshape(1, h),
                               W2b.T, b2b.reshape(1, h),
                               s2, bt2.reshape(1, h),
                               waT[:, 0::2], waT[:, 1::2],
                               c0[:, 0::2], c0[:, 1::2],
                               wbT[:, 0::2], wbT[:, 1::2])

  wcT = wC.T
  wcT_p = jnp.concatenate([wcT[:, 0::2], wcT[:, 1::2]], axis=1)
  w2T = w2p.T
  w2T_p = jnp.concatenate([w2T[0::2], w2T[1::2]], axis=0)

  n_chunks = 5
  ch = n_edges // n_chunks
  outs = []
  for i in range(n_chunks):
    sl = slice(i * ch, (i + 1) * ch)
    ga, gb = _gather_pair(a_tbl, b_tbl, src[sl], dst[sl])
    outs.append(_edge_head_call(ga, gb, edge_attr[sl], wcT_p, w2T_p,
                                b2p, w3p.T, b3p))
  return jnp.concatenate(outs, axis=0)
